# Initial kernel scaffold; baseline (speedup 1.0000x reference)
import jax
import jax.numpy as jnp
from jax.experimental import pallas as pl


def kernel(output, target):
    raise NotImplementedError("write your pallas kernel here")


# speedup vs baseline: 93.8590x; 93.8590x over previous
import functools
import math

import jax
import jax.numpy as jnp
from jax import lax
from jax.experimental import pallas as pl
from jax.experimental.pallas import tpu as pltpu
from jax.experimental.pallas import tpu_sc as plsc

jax.config.update("jax_enable_x64", True)

V = 32000
N = 2048
SMOOTHING = 0.1
CONF = 1.0 - SMOOTHING
SVAL = SMOOTHING / (V - 2)
ROW_TLOGT = (V - 2) * SVAL * math.log(SVAL) + CONF * math.log(CONF)

ROW_BLK = 256
COL_BLK = 6400

_NW = 32
_BPW = N // _NW


def _tc_body(w_ref, x_ref, o_ref):
    @pl.when((pl.program_id(0) == 0) & (pl.program_id(1) == 0))
    def _init():
        o_ref[...] = jnp.zeros_like(o_ref)

    o_ref[...] += jnp.sum(x_ref[...] * w_ref[...])


def _masked_total_sum(output, w):
    return pl.pallas_call(
        _tc_body,
        grid=(N // ROW_BLK, V // COL_BLK),
        in_specs=[
            pl.BlockSpec((ROW_BLK, 1), lambda i, j: (i, jnp.int32(0))),
            pl.BlockSpec((ROW_BLK, COL_BLK), lambda i, j: (i, j)),
        ],
        out_specs=pl.BlockSpec((1, 1), lambda i, j: (jnp.int32(0), jnp.int32(0))),
        out_shape=jax.ShapeDtypeStruct((1, 1), jnp.float32),
        compiler_params=pltpu.CompilerParams(
            dimension_semantics=("arbitrary", "arbitrary")),
    )(w, output)


def _sc_partials(out_flat, tgt32):
    mesh = plsc.VectorSubcoreMesh(core_axis_name="c", subcore_axis_name="s")

    @functools.partial(
        pl.kernel,
        mesh=mesh,
        out_type=jax.ShapeDtypeStruct((_NW, 4, 16), jnp.float32),
        scratch_types=[
            pltpu.VMEM((_BPW,), jnp.int32),
            pltpu.VMEM((2 * _BPW,), jnp.int32),
            pltpu.VMEM((2 * _BPW,), jnp.float32),
            pltpu.VMEM((4, 16), jnp.float32),
            pltpu.SemaphoreType.DMA,
        ],
    )
    def sc_k(x_hbm, t_hbm, o_hbm, t_v, idx_v, g_v, acc_v, sem):
        wid = lax.axis_index("s") * 2 + lax.axis_index("c")
        base = wid * _BPW
        pltpu.sync_copy(t_hbm.at[pl.ds(base, _BPW)], t_v)
        for j in range(_BPW // 16):
            t16 = t_v[pl.ds(j * 16, 16)]
            row16 = base + j * 16 + lax.iota(jnp.int32, 16)
            idx_v[pl.ds(j * 16, 16)] = row16 * V + t16
            idx_v[pl.ds(_BPW + j * 16, 16)] = row16 * V
        pltpu.async_copy(x_hbm.at[idx_v], g_v, sem).wait()
        ones = jnp.ones((16,), jnp.float32)
        zeros = jnp.zeros((16,), jnp.float32)
        accg = zeros
        accz = zeros
        accn = zeros
        for j in range(_BPW // 16):
            t16 = t_v[pl.ds(j * 16, 16)]
            m16 = t16 != jnp.zeros((16,), jnp.int32)
            accg = accg + jnp.where(m16, g_v[pl.ds(j * 16, 16)], zeros)
            accz = accz + jnp.where(m16, g_v[pl.ds(_BPW + j * 16, 16)], zeros)
            accn = accn + jnp.where(m16, ones, zeros)
        acc_v[0] = accg
        acc_v[1] = accz
        acc_v[2] = accn
        acc_v[3] = jnp.zeros((16,), jnp.float32)
        pltpu.sync_copy(acc_v, o_hbm.at[wid])

    return sc_k(out_flat, tgt32)


def kernel(output, target):
    tgt32 = target.astype(jnp.int32)
    w = (tgt32 != 0).astype(jnp.float32)[:, None]
    t_sum = _masked_total_sum(output, w)[0, 0]
    parts = _sc_partials(output.reshape(-1), tgt32)
    g64 = jnp.sum(parts[:, 0, :]).astype(jnp.float64)
    z64 = jnp.sum(parts[:, 1, :]).astype(jnp.float64)
    n64 = jnp.sum(parts[:, 2, :]).astype(jnp.float64)
    t64 = t_sum.astype(jnp.float64)
    return n64 * ROW_TLOGT - SVAL * t64 + SVAL * z64 + (SVAL - CONF) * g64


# speedup vs baseline: 97.2190x; 1.0358x over previous
import functools
import math

import jax
import jax.numpy as jnp
from jax import lax
from jax.experimental import pallas as pl
from jax.experimental.pallas import tpu as pltpu
from jax.experimental.pallas import tpu_sc as plsc

jax.config.update("jax_enable_x64", True)

V = 32000
N = 2048
SMOOTHING = 0.1
CONF = 1.0 - SMOOTHING
SVAL = SMOOTHING / (V - 2)
ROW_TLOGT = (V - 2) * SVAL * math.log(SVAL) + CONF * math.log(CONF)

ROW_BLK = 512
COL_BLK = 6400

_NW = 32
_BPW = N // _NW


def _tc_body(w_ref, x_ref, o_ref):
    @pl.when((pl.program_id(0) == 0) & (pl.program_id(1) == 0))
    def _init():
        o_ref[...] = jnp.zeros_like(o_ref)

    o_ref[...] += jnp.sum(x_ref[...] * w_ref[...])


def _masked_total_sum(output, w):
    return pl.pallas_call(
        _tc_body,
        grid=(N // ROW_BLK, V // COL_BLK),
        in_specs=[
            pl.BlockSpec((ROW_BLK, 1), lambda i, j: (i, jnp.int32(0))),
            pl.BlockSpec((ROW_BLK, COL_BLK), lambda i, j: (i, j)),
        ],
        out_specs=pl.BlockSpec((1, 1), lambda i, j: (jnp.int32(0), jnp.int32(0))),
        out_shape=jax.ShapeDtypeStruct((1, 1), jnp.float32),
        compiler_params=pltpu.CompilerParams(
            dimension_semantics=("arbitrary", "arbitrary")),
    )(w, output)


def _sc_partials(out_flat, tgt32):
    mesh = plsc.VectorSubcoreMesh(core_axis_name="c", subcore_axis_name="s")

    @functools.partial(
        pl.kernel,
        mesh=mesh,
        out_type=jax.ShapeDtypeStruct((_NW, 4, 16), jnp.float32),
        scratch_types=[
            pltpu.VMEM((_BPW,), jnp.int32),
            pltpu.VMEM((2 * _BPW,), jnp.int32),
            pltpu.VMEM((2 * _BPW,), jnp.float32),
            pltpu.VMEM((4, 16), jnp.float32),
            pltpu.SemaphoreType.DMA,
        ],
    )
    def sc_k(x_hbm, t_hbm, o_hbm, t_v, idx_v, g_v, acc_v, sem):
        wid = lax.axis_index("s") * 2 + lax.axis_index("c")
        base = wid * _BPW
        pltpu.sync_copy(t_hbm.at[pl.ds(base, _BPW)], t_v)
        for j in range(_BPW // 16):
            t16 = t_v[pl.ds(j * 16, 16)]
            row16 = base + j * 16 + lax.iota(jnp.int32, 16)
            idx_v[pl.ds(j * 16, 16)] = row16 * V + t16
            idx_v[pl.ds(_BPW + j * 16, 16)] = row16 * V
        pltpu.async_copy(x_hbm.at[idx_v], g_v, sem).wait()
        ones = jnp.ones((16,), jnp.float32)
        zeros = jnp.zeros((16,), jnp.float32)
        accg = zeros
        accz = zeros
        accn = zeros
        for j in range(_BPW // 16):
            t16 = t_v[pl.ds(j * 16, 16)]
            m16 = t16 != jnp.zeros((16,), jnp.int32)
            accg = accg + jnp.where(m16, g_v[pl.ds(j * 16, 16)], zeros)
            accz = accz + jnp.where(m16, g_v[pl.ds(_BPW + j * 16, 16)], zeros)
            accn = accn + jnp.where(m16, ones, zeros)
        acc_v[0] = accg
        acc_v[1] = accz
        acc_v[2] = accn
        acc_v[3] = jnp.zeros((16,), jnp.float32)
        pltpu.sync_copy(acc_v, o_hbm.at[wid])

    return sc_k(out_flat, tgt32)


def kernel(output, target):
    tgt32 = target.astype(jnp.int32)
    w = (tgt32 != 0).astype(jnp.float32)[:, None]
    t_sum = _masked_total_sum(output, w)[0, 0]
    parts = _sc_partials(output.reshape(-1), tgt32)
    g64 = jnp.sum(parts[:, 0, :]).astype(jnp.float64)
    z64 = jnp.sum(parts[:, 1, :]).astype(jnp.float64)
    n64 = jnp.sum(parts[:, 2, :]).astype(jnp.float64)
    t64 = t_sum.astype(jnp.float64)
    return n64 * ROW_TLOGT - SVAL * t64 + SVAL * z64 + (SVAL - CONF) * g64


# speedup vs baseline: 97.7017x; 1.0050x over previous
import functools
import math

import jax
import jax.numpy as jnp
from jax import lax
from jax.experimental import pallas as pl
from jax.experimental.pallas import tpu as pltpu
from jax.experimental.pallas import tpu_sc as plsc

jax.config.update("jax_enable_x64", True)

V = 32000
N = 2048
SMOOTHING = 0.1
CONF = 1.0 - SMOOTHING
SVAL = SMOOTHING / (V - 2)
ROW_TLOGT = (V - 2) * SVAL * math.log(SVAL) + CONF * math.log(CONF)

ROW_BLK = 2048
COL_BLK = 3200

_NW = 32
_BPW = N // _NW


def _tc_body(w_ref, x_ref, o_ref):
    @pl.when((pl.program_id(0) == 0) & (pl.program_id(1) == 0))
    def _init():
        o_ref[...] = jnp.zeros_like(o_ref)

    o_ref[...] += jnp.sum(x_ref[...] * w_ref[...])


def _masked_total_sum(output, w):
    return pl.pallas_call(
        _tc_body,
        grid=(N // ROW_BLK, V // COL_BLK),
        in_specs=[
            pl.BlockSpec((ROW_BLK, 1), lambda i, j: (i, jnp.int32(0))),
            pl.BlockSpec((ROW_BLK, COL_BLK), lambda i, j: (i, j)),
        ],
        out_specs=pl.BlockSpec((1, 1), lambda i, j: (jnp.int32(0), jnp.int32(0))),
        out_shape=jax.ShapeDtypeStruct((1, 1), jnp.float32),
        compiler_params=pltpu.CompilerParams(
            dimension_semantics=("arbitrary", "arbitrary")),
    )(w, output)


def _sc_partials(out_flat, tgt32):
    mesh = plsc.VectorSubcoreMesh(core_axis_name="c", subcore_axis_name="s")

    @functools.partial(
        pl.kernel,
        mesh=mesh,
        out_type=jax.ShapeDtypeStruct((_NW, 4, 16), jnp.float32),
        scratch_types=[
            pltpu.VMEM((_BPW,), jnp.int32),
            pltpu.VMEM((2 * _BPW,), jnp.int32),
            pltpu.VMEM((2 * _BPW,), jnp.float32),
            pltpu.VMEM((4, 16), jnp.float32),
            pltpu.SemaphoreType.DMA,
        ],
    )
    def sc_k(x_hbm, t_hbm, o_hbm, t_v, idx_v, g_v, acc_v, sem):
        wid = lax.axis_index("s") * 2 + lax.axis_index("c")
        base = wid * _BPW
        pltpu.sync_copy(t_hbm.at[pl.ds(base, _BPW)], t_v)
        for j in range(_BPW // 16):
            t16 = t_v[pl.ds(j * 16, 16)]
            row16 = base + j * 16 + lax.iota(jnp.int32, 16)
            idx_v[pl.ds(j * 16, 16)] = row16 * V + t16
            idx_v[pl.ds(_BPW + j * 16, 16)] = row16 * V
        pltpu.async_copy(x_hbm.at[idx_v], g_v, sem).wait()
        ones = jnp.ones((16,), jnp.float32)
        zeros = jnp.zeros((16,), jnp.float32)
        accg = zeros
        accz = zeros
        accn = zeros
        for j in range(_BPW // 16):
            t16 = t_v[pl.ds(j * 16, 16)]
            m16 = t16 != jnp.zeros((16,), jnp.int32)
            accg = accg + jnp.where(m16, g_v[pl.ds(j * 16, 16)], zeros)
            accz = accz + jnp.where(m16, g_v[pl.ds(_BPW + j * 16, 16)], zeros)
            accn = accn + jnp.where(m16, ones, zeros)
        acc_v[0] = accg
        acc_v[1] = accz
        acc_v[2] = accn
        acc_v[3] = jnp.zeros((16,), jnp.float32)
        pltpu.sync_copy(acc_v, o_hbm.at[wid])

    return sc_k(out_flat, tgt32)


def kernel(output, target):
    tgt32 = target.astype(jnp.int32)
    w = (tgt32 != 0).astype(jnp.float32)[:, None]
    t_sum = _masked_total_sum(output, w)[0, 0]
    parts = _sc_partials(output.reshape(-1), tgt32)
    g64 = jnp.sum(parts[:, 0, :]).astype(jnp.float64)
    z64 = jnp.sum(parts[:, 1, :]).astype(jnp.float64)
    n64 = jnp.sum(parts[:, 2, :]).astype(jnp.float64)
    t64 = t_sum.astype(jnp.float64)
    return n64 * ROW_TLOGT - SVAL * t64 + SVAL * z64 + (SVAL - CONF) * g64


# speedup vs baseline: 100.6860x; 1.0305x over previous
import functools
import math

import jax
import jax.numpy as jnp
from jax import lax
from jax.experimental import pallas as pl
from jax.experimental.pallas import tpu as pltpu
from jax.experimental.pallas import tpu_sc as plsc

jax.config.update("jax_enable_x64", True)

V = 32000
N = 2048
SMOOTHING = 0.1
CONF = 1.0 - SMOOTHING
SVAL = SMOOTHING / (V - 2)
ROW_TLOGT = (V - 2) * SVAL * math.log(SVAL) + CONF * math.log(CONF)

ROW_BLK = 512
COL_BLK = 3200

_NW = 32
_BPW = N // _NW


def _tc_body(w_ref, x1_ref, x2_ref, o_ref):
    @pl.when((pl.program_id(0) == 0) & (pl.program_id(1) == 0))
    def _init():
        o_ref[...] = jnp.zeros_like(o_ref)

    o_ref[...] += jnp.sum((x1_ref[...] + x2_ref[...]) * w_ref[...])


def _masked_total_sum(output, w):
    return pl.pallas_call(
        _tc_body,
        grid=(N // ROW_BLK, (V // 2) // COL_BLK),
        in_specs=[
            pl.BlockSpec((ROW_BLK, 1), lambda i, j: (i, jnp.int32(0))),
            pl.BlockSpec((ROW_BLK, COL_BLK), lambda i, j: (i, j)),
            pl.BlockSpec((ROW_BLK, COL_BLK),
                         lambda i, j: (i, j + (V // 2) // COL_BLK)),
        ],
        out_specs=pl.BlockSpec((1, 1), lambda i, j: (jnp.int32(0), jnp.int32(0))),
        out_shape=jax.ShapeDtypeStruct((1, 1), jnp.float32),
        compiler_params=pltpu.CompilerParams(
            dimension_semantics=("arbitrary", "arbitrary")),
    )(w, output, output)


def _sc_partials(out_flat, tgt32):
    mesh = plsc.VectorSubcoreMesh(core_axis_name="c", subcore_axis_name="s")

    @functools.partial(
        pl.kernel,
        mesh=mesh,
        out_type=jax.ShapeDtypeStruct((_NW, 4, 16), jnp.float32),
        scratch_types=[
            pltpu.VMEM((_BPW,), jnp.int32),
            pltpu.VMEM((2 * _BPW,), jnp.int32),
            pltpu.VMEM((2 * _BPW,), jnp.float32),
            pltpu.VMEM((4, 16), jnp.float32),
            pltpu.SemaphoreType.DMA,
        ],
    )
    def sc_k(x_hbm, t_hbm, o_hbm, t_v, idx_v, g_v, acc_v, sem):
        wid = lax.axis_index("s") * 2 + lax.axis_index("c")
        base = wid * _BPW
        pltpu.sync_copy(t_hbm.at[pl.ds(base, _BPW)], t_v)
        for j in range(_BPW // 16):
            t16 = t_v[pl.ds(j * 16, 16)]
            row16 = base + j * 16 + lax.iota(jnp.int32, 16)
            idx_v[pl.ds(j * 16, 16)] = row16 * V + t16
            idx_v[pl.ds(_BPW + j * 16, 16)] = row16 * V
        pltpu.async_copy(x_hbm.at[idx_v], g_v, sem).wait()
        ones = jnp.ones((16,), jnp.float32)
        zeros = jnp.zeros((16,), jnp.float32)
        accg = zeros
        accz = zeros
        accn = zeros
        for j in range(_BPW // 16):
            t16 = t_v[pl.ds(j * 16, 16)]
            m16 = t16 != jnp.zeros((16,), jnp.int32)
            accg = accg + jnp.where(m16, g_v[pl.ds(j * 16, 16)], zeros)
            accz = accz + jnp.where(m16, g_v[pl.ds(_BPW + j * 16, 16)], zeros)
            accn = accn + jnp.where(m16, ones, zeros)
        acc_v[0] = accg
        acc_v[1] = accz
        acc_v[2] = accn
        acc_v[3] = jnp.zeros((16,), jnp.float32)
        pltpu.sync_copy(acc_v, o_hbm.at[wid])

    return sc_k(out_flat, tgt32)


def kernel(output, target):
    tgt32 = target.astype(jnp.int32)
    w = (tgt32 != 0).astype(jnp.float32)[:, None]
    t_sum = _masked_total_sum(output, w)[0, 0]
    parts = _sc_partials(output.reshape(-1), tgt32)
    g64 = jnp.sum(parts[:, 0, :]).astype(jnp.float64)
    z64 = jnp.sum(parts[:, 1, :]).astype(jnp.float64)
    n64 = jnp.sum(parts[:, 2, :]).astype(jnp.float64)
    t64 = t_sum.astype(jnp.float64)
    return n64 * ROW_TLOGT - SVAL * t64 + SVAL * z64 + (SVAL - CONF) * g64
